# baseline (speedup 1.0000x reference)
import functools

import jax
import jax.numpy as jnp
from jax import lax
from jax.experimental import pallas as pl
from jax.experimental.pallas import tpu as pltpu
from jax.experimental.pallas import tpu_sc as plsc

HIDDEN = 768
EPS = 1e-12
GATHER_W = 64


def _sc_gather(table, ids):
    n = ids.shape[0]
    mesh = plsc.VectorSubcoreMesh(core_axis_name="core", subcore_axis_name="subcore")
    nw = mesh.num_cores * mesh.num_subcores
    per_w = n // nw
    ng = per_w // GATHER_W

    @functools.partial(
        pl.kernel,
        out_type=jax.ShapeDtypeStruct((n, HIDDEN), table.dtype),
        mesh=mesh,
        scratch_types=[
            pltpu.VMEM((per_w,), jnp.int32),
            pltpu.VMEM((2, GATHER_W, HIDDEN), jnp.float32),
            pltpu.SemaphoreType.DMA,
            pltpu.SemaphoreType.DMA,
            pltpu.SemaphoreType.DMA,
        ],
    )
    def gather_kernel(table_hbm, ids_hbm, out_hbm, idx_v, rows_v, gsem,
                      wsem0, wsem1):
        wid = lax.axis_index("subcore") * mesh.num_cores + lax.axis_index("core")
        base = wid * per_w
        pltpu.sync_copy(ids_hbm.at[pl.ds(base, per_w)], idx_v)
        wsems = (wsem0, wsem1)
        writes = [None, None]
        for g in range(ng):
            b = g & 1
            if writes[b] is not None:
                writes[b].wait()
            pltpu.async_copy(
                table_hbm.at[idx_v.at[pl.ds(g * GATHER_W, GATHER_W)]],
                rows_v.at[b], gsem,
            ).wait()
            writes[b] = pltpu.async_copy(
                rows_v.at[b], out_hbm.at[pl.ds(base + g * GATHER_W, GATHER_W)],
                wsems[b])
        for w in writes:
            if w is not None:
                w.wait()

    return gather_kernel(table, ids)


def _tc_add_ln_body(x_ref, tt_ref, pos_ref, type_ref, gamma_ref, beta_ref, o_ref):
    x = x_ref[...]
    tt = tt_ref[:, :1]
    pos = pos_ref[...]
    t0 = type_ref[0, :]
    t1 = type_ref[1, :]
    te = tt * (t1 - t0)[None, :] + t0[None, :]
    e = x + pos + te
    mean = jnp.mean(e, axis=-1, keepdims=True)
    c = e - mean
    var = jnp.mean(c * c, axis=-1, keepdims=True)
    normed = c * lax.rsqrt(var + EPS)
    o_ref[...] = normed * gamma_ref[0, :] + beta_ref[0, :]


def _acc_add_ln_body(acc_ref, x_ref, tt_ref, pos_ref, type_ref, gamma_ref,
                     beta_ref, o_ref):
    del acc_ref
    _tc_add_ln_body(x_ref, tt_ref, pos_ref, type_ref, gamma_ref, beta_ref,
                    o_ref)


def _tc_add_ln_chunk(acc, g_c, tt8, position_embeddings,
                     token_type_embeddings, ln_gamma, ln_beta,
                     n_total, chunk_rows, row0, rows):
    pos_blocks = max(position_embeddings.shape[0] // rows, 1)
    blk0 = row0 // rows
    grid = (chunk_rows // rows,)
    out_spec = pl.BlockSpec((rows, HIDDEN), lambda i: (blk0 + i, 0))
    out_shape = jax.ShapeDtypeStruct((n_total, HIDDEN), jnp.float32)
    common_specs = [
        pl.BlockSpec((rows, HIDDEN), lambda i: (i, 0)),
        pl.BlockSpec((rows, 8), lambda i: (blk0 + i, 0)),
        pl.BlockSpec((rows, HIDDEN), lambda i: (i % pos_blocks, 0)),
        pl.BlockSpec((2, HIDDEN), lambda i: (0, 0)),
        pl.BlockSpec((1, HIDDEN), lambda i: (0, 0)),
        pl.BlockSpec((1, HIDDEN), lambda i: (0, 0)),
    ]
    args = (g_c, tt8, position_embeddings, token_type_embeddings,
            ln_gamma.reshape(1, HIDDEN), ln_beta.reshape(1, HIDDEN))
    if acc is None:
        return pl.pallas_call(
            _tc_add_ln_body,
            grid=grid,
            in_specs=common_specs,
            out_specs=out_spec,
            out_shape=out_shape,
        )(*args)
    return pl.pallas_call(
        _acc_add_ln_body,
        grid=grid,
        in_specs=[pl.BlockSpec(memory_space=pl.MemorySpace.ANY)] + common_specs,
        out_specs=out_spec,
        out_shape=out_shape,
        input_output_aliases={0: 0},
    )(acc, *args)


CHUNK_SIZES = (2048, 6144, 8192, 8192, 6144, 2048)
TC_ROWS = 1024


def kernel(input_ids, token_type_ids, position_ids, word_embeddings,
           position_embeddings, token_type_embeddings, ln_gamma, ln_beta):
    batch, seq = input_ids.shape
    n = batch * seq
    ids = input_ids.astype(jnp.int32).reshape(n)
    tt8 = jnp.broadcast_to(
        token_type_ids.astype(jnp.float32).reshape(n, 1), (n, 8))

    pos = position_embeddings
    if TC_ROWS > seq:
        pos = jnp.tile(position_embeddings, (TC_ROWS // seq, 1))
    assert sum(CHUNK_SIZES) == n
    starts = [sum(CHUNK_SIZES[:c]) for c in range(len(CHUNK_SIZES))]
    gs = []
    for c, nc in enumerate(CHUNK_SIZES):
        ids_c = lax.slice(ids, (starts[c],), (starts[c] + nc,))
        gs.append(_sc_gather(word_embeddings, ids_c))
    acc = None
    for c, nc in enumerate(CHUNK_SIZES):
        acc = _tc_add_ln_chunk(acc, gs[c], tt8, pos,
                               token_type_embeddings, ln_gamma, ln_beta,
                               n, nc, starts[c], TC_ROWS)
    return acc.reshape(batch, seq, HIDDEN)


# speedup vs baseline: 1.0337x; 1.0337x over previous
import functools

import jax
import jax.numpy as jnp
from jax import lax
from jax.experimental import pallas as pl
from jax.experimental.pallas import tpu as pltpu
from jax.experimental.pallas import tpu_sc as plsc

HIDDEN = 768
EPS = 1e-12
GATHER_W = 64


def _sc_gather(table, ids):
    n = ids.shape[0]
    mesh = plsc.VectorSubcoreMesh(core_axis_name="core", subcore_axis_name="subcore")
    nw = mesh.num_cores * mesh.num_subcores
    per_w = n // nw
    ng = per_w // GATHER_W

    @functools.partial(
        pl.kernel,
        out_type=jax.ShapeDtypeStruct((n, HIDDEN), table.dtype),
        mesh=mesh,
        scratch_types=[
            pltpu.VMEM((per_w,), jnp.int32),
            pltpu.VMEM((2, GATHER_W, HIDDEN), jnp.float32),
            pltpu.SemaphoreType.DMA,
            pltpu.SemaphoreType.DMA,
            pltpu.SemaphoreType.DMA,
        ],
    )
    def gather_kernel(table_hbm, ids_hbm, out_hbm, idx_v, rows_v, gsem,
                      wsem0, wsem1):
        wid = lax.axis_index("subcore") * mesh.num_cores + lax.axis_index("core")
        base = wid * per_w
        pltpu.sync_copy(ids_hbm.at[pl.ds(base, per_w)], idx_v)
        wsems = (wsem0, wsem1)
        writes = [None, None]
        for g in range(ng):
            b = g & 1
            if writes[b] is not None:
                writes[b].wait()
            pltpu.async_copy(
                table_hbm.at[idx_v.at[pl.ds(g * GATHER_W, GATHER_W)]],
                rows_v.at[b], gsem,
            ).wait()
            writes[b] = pltpu.async_copy(
                rows_v.at[b], out_hbm.at[pl.ds(base + g * GATHER_W, GATHER_W)],
                wsems[b])
        for w in writes:
            if w is not None:
                w.wait()

    return gather_kernel(table, ids)


def _tc_add_ln_body(x_ref, tt_ref, pos_ref, type_ref, gamma_ref, beta_ref, o_ref):
    x = x_ref[...]
    tt = tt_ref[:, :1].astype(jnp.float32)
    pos = pos_ref[...]
    t0 = type_ref[0, :]
    t1 = type_ref[1, :]
    te = tt * (t1 - t0)[None, :] + t0[None, :]
    e = x + pos + te
    mean = jnp.mean(e, axis=-1, keepdims=True)
    c = e - mean
    var = jnp.mean(c * c, axis=-1, keepdims=True)
    normed = c * lax.rsqrt(var + EPS)
    o_ref[...] = normed * gamma_ref[0, :] + beta_ref[0, :]


def _acc_add_ln_body(acc_ref, x_ref, tt_ref, pos_ref, type_ref, gamma_ref,
                     beta_ref, o_ref):
    del acc_ref
    _tc_add_ln_body(x_ref, tt_ref, pos_ref, type_ref, gamma_ref, beta_ref,
                    o_ref)


def _tc_add_ln_chunk(acc, g_c, tt8, position_embeddings,
                     token_type_embeddings, ln_gamma, ln_beta,
                     n_total, chunk_rows, row0, rows):
    pos_blocks = max(position_embeddings.shape[0] // rows, 1)
    blk0 = row0 // rows
    grid = (chunk_rows // rows,)
    out_spec = pl.BlockSpec((rows, HIDDEN), lambda i: (blk0 + i, 0))
    out_shape = jax.ShapeDtypeStruct((n_total, HIDDEN), jnp.float32)
    common_specs = [
        pl.BlockSpec((rows, HIDDEN), lambda i: (i, 0)),
        pl.BlockSpec((rows, 8), lambda i: (blk0 + i, 0)),
        pl.BlockSpec((rows, HIDDEN), lambda i: (i % pos_blocks, 0)),
        pl.BlockSpec((2, HIDDEN), lambda i: (0, 0)),
        pl.BlockSpec((1, HIDDEN), lambda i: (0, 0)),
        pl.BlockSpec((1, HIDDEN), lambda i: (0, 0)),
    ]
    args = (g_c, tt8, position_embeddings, token_type_embeddings,
            ln_gamma.reshape(1, HIDDEN), ln_beta.reshape(1, HIDDEN))
    if acc is None:
        return pl.pallas_call(
            _tc_add_ln_body,
            grid=grid,
            in_specs=common_specs,
            out_specs=out_spec,
            out_shape=out_shape,
        )(*args)
    return pl.pallas_call(
        _acc_add_ln_body,
        grid=grid,
        in_specs=[pl.BlockSpec(memory_space=pl.MemorySpace.ANY)] + common_specs,
        out_specs=out_spec,
        out_shape=out_shape,
        input_output_aliases={0: 0},
    )(acc, *args)


CHUNK_SIZES = (2048, 6144, 8192, 8192, 6144, 2048)
TC_ROWS = 1024


def kernel(input_ids, token_type_ids, position_ids, word_embeddings,
           position_embeddings, token_type_embeddings, ln_gamma, ln_beta):
    batch, seq = input_ids.shape
    n = batch * seq
    ids = input_ids.astype(jnp.int32).reshape(n)
    tt8 = jnp.broadcast_to(
        token_type_ids.astype(jnp.int8).reshape(n, 1), (n, 8))

    pos = position_embeddings
    if TC_ROWS > seq:
        pos = jnp.tile(position_embeddings, (TC_ROWS // seq, 1))
    assert sum(CHUNK_SIZES) == n
    starts = [sum(CHUNK_SIZES[:c]) for c in range(len(CHUNK_SIZES))]
    gs = []
    for c, nc in enumerate(CHUNK_SIZES):
        ids_c = lax.slice(ids, (starts[c],), (starts[c] + nc,))
        gs.append(_sc_gather(word_embeddings, ids_c))
    acc = None
    for c, nc in enumerate(CHUNK_SIZES):
        acc = _tc_add_ln_chunk(acc, gs[c], tt8, pos,
                               token_type_embeddings, ln_gamma, ln_beta,
                               n, nc, starts[c], TC_ROWS)
    return acc.reshape(batch, seq, HIDDEN)


# speedup vs baseline: 1.0441x; 1.0101x over previous
import functools

import jax
import jax.numpy as jnp
from jax import lax
from jax.experimental import pallas as pl
from jax.experimental.pallas import tpu as pltpu
from jax.experimental.pallas import tpu_sc as plsc

HIDDEN = 768
EPS = 1e-12
GATHER_W = 64


def _sc_gather(table, ids):
    n = ids.shape[0]
    mesh = plsc.VectorSubcoreMesh(core_axis_name="core", subcore_axis_name="subcore")
    nw = mesh.num_cores * mesh.num_subcores
    per_w = n // nw
    ng = per_w // GATHER_W

    @functools.partial(
        pl.kernel,
        out_type=jax.ShapeDtypeStruct((n, HIDDEN), table.dtype),
        mesh=mesh,
        scratch_types=[
            pltpu.VMEM((per_w,), jnp.int32),
            pltpu.VMEM((2, GATHER_W, HIDDEN), jnp.float32),
            pltpu.SemaphoreType.DMA,
            pltpu.SemaphoreType.DMA,
            pltpu.SemaphoreType.DMA,
        ],
    )
    def gather_kernel(table_hbm, ids_hbm, out_hbm, idx_v, rows_v, gsem,
                      wsem0, wsem1):
        wid = lax.axis_index("subcore") * mesh.num_cores + lax.axis_index("core")
        base = wid * per_w
        pltpu.sync_copy(ids_hbm.at[pl.ds(base, per_w)], idx_v)
        wsems = (wsem0, wsem1)
        writes = [None, None]
        for g in range(ng):
            b = g & 1
            if writes[b] is not None:
                writes[b].wait()
            pltpu.async_copy(
                table_hbm.at[idx_v.at[pl.ds(g * GATHER_W, GATHER_W)]],
                rows_v.at[b], gsem,
            ).wait()
            writes[b] = pltpu.async_copy(
                rows_v.at[b], out_hbm.at[pl.ds(base + g * GATHER_W, GATHER_W)],
                wsems[b])
        for w in writes:
            if w is not None:
                w.wait()

    return gather_kernel(table, ids)


def _tc_add_ln_body(x_ref, tt_ref, pos_ref, type_ref, gamma_ref, beta_ref, o_ref):
    x = x_ref[...]
    tt = tt_ref[:, :1].astype(jnp.float32)
    pos = pos_ref[...]
    t0 = type_ref[0, :]
    t1 = type_ref[1, :]
    te = tt * (t1 - t0)[None, :] + t0[None, :]
    e = x + pos + te
    mean = jnp.mean(e, axis=-1, keepdims=True)
    c = e - mean
    var = jnp.mean(c * c, axis=-1, keepdims=True)
    normed = c * lax.rsqrt(var + EPS)
    o_ref[...] = normed * gamma_ref[0, :] + beta_ref[0, :]


def _acc_add_ln_body(acc_ref, x_ref, tt_ref, pos_ref, type_ref, gamma_ref,
                     beta_ref, o_ref):
    del acc_ref
    _tc_add_ln_body(x_ref, tt_ref, pos_ref, type_ref, gamma_ref, beta_ref,
                    o_ref)


def _tc_add_ln_chunk(acc, g_c, tt8, position_embeddings,
                     token_type_embeddings, ln_gamma, ln_beta,
                     n_total, chunk_rows, row0, rows):
    pos_blocks = max(position_embeddings.shape[0] // rows, 1)
    blk0 = row0 // rows
    grid = (chunk_rows // rows,)
    out_spec = pl.BlockSpec((rows, HIDDEN), lambda i: (blk0 + i, 0))
    out_shape = jax.ShapeDtypeStruct((n_total, HIDDEN), jnp.float32)
    common_specs = [
        pl.BlockSpec((rows, HIDDEN), lambda i: (i, 0)),
        pl.BlockSpec((rows, 8), lambda i: (blk0 + i, 0)),
        pl.BlockSpec((rows, HIDDEN), lambda i: (i % pos_blocks, 0)),
        pl.BlockSpec((2, HIDDEN), lambda i: (0, 0)),
        pl.BlockSpec((1, HIDDEN), lambda i: (0, 0)),
        pl.BlockSpec((1, HIDDEN), lambda i: (0, 0)),
    ]
    args = (g_c, tt8, position_embeddings, token_type_embeddings,
            ln_gamma.reshape(1, HIDDEN), ln_beta.reshape(1, HIDDEN))
    if acc is None:
        return pl.pallas_call(
            _tc_add_ln_body,
            grid=grid,
            in_specs=common_specs,
            out_specs=out_spec,
            out_shape=out_shape,
        )(*args)
    return pl.pallas_call(
        _acc_add_ln_body,
        grid=grid,
        in_specs=[pl.BlockSpec(memory_space=pl.MemorySpace.ANY)] + common_specs,
        out_specs=out_spec,
        out_shape=out_shape,
        input_output_aliases={0: 0},
    )(acc, *args)


CHUNK_SIZES = (2048, 8192, 10240, 8192, 4096)
TC_ROWS = 1024


def kernel(input_ids, token_type_ids, position_ids, word_embeddings,
           position_embeddings, token_type_embeddings, ln_gamma, ln_beta):
    batch, seq = input_ids.shape
    n = batch * seq
    ids = input_ids.astype(jnp.int32).reshape(n)
    tt8 = jnp.broadcast_to(
        token_type_ids.astype(jnp.int8).reshape(n, 1), (n, 8))

    pos = position_embeddings
    if TC_ROWS > seq:
        pos = jnp.tile(position_embeddings, (TC_ROWS // seq, 1))
    assert sum(CHUNK_SIZES) == n
    starts = [sum(CHUNK_SIZES[:c]) for c in range(len(CHUNK_SIZES))]
    gs = []
    for c, nc in enumerate(CHUNK_SIZES):
        ids_c = lax.slice(ids, (starts[c],), (starts[c] + nc,))
        gs.append(_sc_gather(word_embeddings, ids_c))
    acc = None
    for c, nc in enumerate(CHUNK_SIZES):
        acc = _tc_add_ln_chunk(acc, gs[c], tt8, pos,
                               token_type_embeddings, ln_gamma, ln_beta,
                               n, nc, starts[c], TC_ROWS)
    return acc.reshape(batch, seq, HIDDEN)


# speedup vs baseline: 1.0507x; 1.0063x over previous
import functools

import jax
import jax.numpy as jnp
from jax import lax
from jax.experimental import pallas as pl
from jax.experimental.pallas import tpu as pltpu
from jax.experimental.pallas import tpu_sc as plsc

HIDDEN = 768
EPS = 1e-12
GATHER_W = 64


def _sc_gather(table, ids):
    n = ids.shape[0]
    mesh = plsc.VectorSubcoreMesh(core_axis_name="core", subcore_axis_name="subcore")
    nw = mesh.num_cores * mesh.num_subcores
    per_w = n // nw
    ng = per_w // GATHER_W

    @functools.partial(
        pl.kernel,
        out_type=jax.ShapeDtypeStruct((n, HIDDEN), table.dtype),
        mesh=mesh,
        scratch_types=[
            pltpu.VMEM((per_w,), jnp.int32),
            pltpu.VMEM((2, GATHER_W, HIDDEN), jnp.float32),
            pltpu.SemaphoreType.DMA,
            pltpu.SemaphoreType.DMA,
            pltpu.SemaphoreType.DMA,
        ],
    )
    def gather_kernel(table_hbm, ids_hbm, out_hbm, idx_v, rows_v, gsem,
                      wsem0, wsem1):
        wid = lax.axis_index("subcore") * mesh.num_cores + lax.axis_index("core")
        base = wid * per_w
        pltpu.sync_copy(ids_hbm.at[pl.ds(base, per_w)], idx_v)
        wsems = (wsem0, wsem1)
        writes = [None, None]
        for g in range(ng):
            b = g & 1
            if writes[b] is not None:
                writes[b].wait()
            pltpu.async_copy(
                table_hbm.at[idx_v.at[pl.ds(g * GATHER_W, GATHER_W)]],
                rows_v.at[b], gsem,
            ).wait()
            writes[b] = pltpu.async_copy(
                rows_v.at[b], out_hbm.at[pl.ds(base + g * GATHER_W, GATHER_W)],
                wsems[b])
        for w in writes:
            if w is not None:
                w.wait()

    return gather_kernel(table, ids)


def _tc_add_ln_body(x_ref, tt_ref, pos_ref, type_ref, gamma_ref, beta_ref, o_ref):
    x = x_ref[...]
    tt = tt_ref[:, :1].astype(jnp.float32)
    pos = pos_ref[...]
    t0 = type_ref[0, :]
    t1 = type_ref[1, :]
    te = tt * (t1 - t0)[None, :] + t0[None, :]
    e = x + pos + te
    mean = jnp.mean(e, axis=-1, keepdims=True)
    c = e - mean
    var = jnp.mean(c * c, axis=-1, keepdims=True)
    normed = c * lax.rsqrt(var + EPS)
    o_ref[...] = normed * gamma_ref[0, :] + beta_ref[0, :]


def _acc_add_ln_body(acc_ref, x_ref, tt_ref, pos_ref, type_ref, gamma_ref,
                     beta_ref, o_ref):
    del acc_ref
    _tc_add_ln_body(x_ref, tt_ref, pos_ref, type_ref, gamma_ref, beta_ref,
                    o_ref)


def _tc_add_ln_chunk(acc, g_c, tt8, position_embeddings,
                     token_type_embeddings, ln_gamma, ln_beta,
                     n_total, chunk_rows, row0, rows):
    pos_blocks = max(position_embeddings.shape[0] // rows, 1)
    blk0 = row0 // rows
    grid = (chunk_rows // rows,)
    out_spec = pl.BlockSpec((rows, HIDDEN), lambda i: (blk0 + i, 0))
    out_shape = jax.ShapeDtypeStruct((n_total, HIDDEN), jnp.float32)
    common_specs = [
        pl.BlockSpec((rows, HIDDEN), lambda i: (i, 0)),
        pl.BlockSpec((rows, 8), lambda i: (blk0 + i, 0)),
        pl.BlockSpec((rows, HIDDEN), lambda i: (i % pos_blocks, 0)),
        pl.BlockSpec((2, HIDDEN), lambda i: (0, 0)),
        pl.BlockSpec((1, HIDDEN), lambda i: (0, 0)),
        pl.BlockSpec((1, HIDDEN), lambda i: (0, 0)),
    ]
    args = (g_c, tt8, position_embeddings, token_type_embeddings,
            ln_gamma.reshape(1, HIDDEN), ln_beta.reshape(1, HIDDEN))
    if acc is None:
        return pl.pallas_call(
            _tc_add_ln_body,
            grid=grid,
            in_specs=common_specs,
            out_specs=out_spec,
            out_shape=out_shape,
        )(*args)
    return pl.pallas_call(
        _acc_add_ln_body,
        grid=grid,
        in_specs=[pl.BlockSpec(memory_space=pl.MemorySpace.ANY)] + common_specs,
        out_specs=out_spec,
        out_shape=out_shape,
        input_output_aliases={0: 0},
    )(acc, *args)


CHUNK_SIZES = (2048, 10240, 10240, 8192, 2048)
TC_ROWS = 1024


def kernel(input_ids, token_type_ids, position_ids, word_embeddings,
           position_embeddings, token_type_embeddings, ln_gamma, ln_beta):
    batch, seq = input_ids.shape
    n = batch * seq
    ids = input_ids.astype(jnp.int32).reshape(n)
    tt8 = jnp.broadcast_to(
        token_type_ids.astype(jnp.int8).reshape(n, 1), (n, 8))

    pos = position_embeddings
    if TC_ROWS > seq:
        pos = jnp.tile(position_embeddings, (TC_ROWS // seq, 1))
    assert sum(CHUNK_SIZES) == n
    starts = [sum(CHUNK_SIZES[:c]) for c in range(len(CHUNK_SIZES))]
    gs = []
    for c, nc in enumerate(CHUNK_SIZES):
        ids_c = lax.slice(ids, (starts[c],), (starts[c] + nc,))
        gs.append(_sc_gather(word_embeddings, ids_c))
    acc = None
    for c, nc in enumerate(CHUNK_SIZES):
        acc = _tc_add_ln_chunk(acc, gs[c], tt8, pos,
                               token_type_embeddings, ln_gamma, ln_beta,
                               n, nc, starts[c], TC_ROWS)
    return acc.reshape(batch, seq, HIDDEN)
